# Initial kernel scaffold; baseline (speedup 1.0000x reference)
import jax
import jax.numpy as jnp
from jax.experimental import pallas as pl


def kernel(x):
    raise NotImplementedError("write your pallas kernel here")


# speedup vs baseline: 6.9227x; 6.9227x over previous
import jax
import jax.numpy as jnp
from jax import lax
from jax.experimental import pallas as pl
from jax.experimental.pallas import tpu as pltpu

_BLK = 2048
_SUB = 256


def _cumsum_body(x_ref, o_ref, carry_ref):
    i = pl.program_id(0)

    @pl.when(i == 0)
    def _init():
        carry_ref[...] = jnp.zeros_like(carry_ref)

    row = lax.broadcasted_iota(jnp.int32, (_SUB, _SUB), 0)
    col = lax.broadcasted_iota(jnp.int32, (_SUB, _SUB), 1)
    tri = (row <= col).astype(jnp.float32)

    offs = carry_ref[...]
    for k in range(_BLK // _SUB):
        sl = pl.ds(k * _SUB, _SUB)
        y = lax.dot(
            x_ref[:, sl],
            tri,
            precision=lax.Precision.HIGHEST,
            preferred_element_type=jnp.float32,
        )
        o_ref[:, sl] = y + offs
        offs = offs + y[:, -1:]
    carry_ref[...] = offs


def kernel(x):
    m, n = x.shape
    grid = (n // _BLK,)
    return pl.pallas_call(
        _cumsum_body,
        grid=grid,
        in_specs=[pl.BlockSpec((m, _BLK), lambda i: (0, i))],
        out_specs=pl.BlockSpec((m, _BLK), lambda i: (0, i)),
        out_shape=jax.ShapeDtypeStruct((m, n), x.dtype),
        scratch_shapes=[pltpu.VMEM((m, 1), jnp.float32)],
    )(x)


# speedup vs baseline: 7.0310x; 1.0156x over previous
import jax
import jax.numpy as jnp
from jax import lax
from jax.experimental import pallas as pl
from jax.experimental.pallas import tpu as pltpu

_BLK = 2048
_SUB = 128


def _cumsum_body(x_ref, o_ref, carry_ref):
    i = pl.program_id(0)

    @pl.when(i == 0)
    def _init():
        carry_ref[...] = jnp.zeros_like(carry_ref)

    row = lax.broadcasted_iota(jnp.int32, (_SUB, _SUB), 0)
    col = lax.broadcasted_iota(jnp.int32, (_SUB, _SUB), 1)
    tri = (row <= col).astype(jnp.bfloat16)

    offs = carry_ref[...]
    for k in range(_BLK // _SUB):
        sl = pl.ds(k * _SUB, _SUB)
        y = lax.dot(
            x_ref[:, sl].astype(jnp.bfloat16),
            tri,
            preferred_element_type=jnp.float32,
        )
        o_ref[:, sl] = y + offs
        offs = offs + y[:, -1:]
    carry_ref[...] = offs


def kernel(x):
    m, n = x.shape
    grid = (n // _BLK,)
    return pl.pallas_call(
        _cumsum_body,
        grid=grid,
        in_specs=[pl.BlockSpec((m, _BLK), lambda i: (0, i))],
        out_specs=pl.BlockSpec((m, _BLK), lambda i: (0, i)),
        out_shape=jax.ShapeDtypeStruct((m, n), x.dtype),
        scratch_shapes=[pltpu.VMEM((m, 1), jnp.float32)],
    )(x)


# speedup vs baseline: 8.2029x; 1.1667x over previous
import jax
import jax.numpy as jnp
import numpy as np
from jax import lax
from jax.experimental import pallas as pl
from jax.experimental.pallas import tpu as pltpu

_BLK = 2048
_SUB = 128
_K = _BLK // _SUB


def _cumsum_body(x_ref, tri_ref, b_ref, o_ref, carry_ref):
    i = pl.program_id(0)

    @pl.when(i == 0)
    def _init():
        carry_ref[...] = jnp.zeros_like(carry_ref)

    xb = x_ref[...].astype(jnp.bfloat16)
    pex = lax.dot(xb, b_ref[...], preferred_element_type=jnp.float32)
    offs = pex + carry_ref[...]
    for k in range(_K):
        lo, hi = k * _SUB, (k + 1) * _SUB
        y = lax.dot(
            xb[:, lo:hi], tri_ref[...], preferred_element_type=jnp.float32
        )
        o_ref[:, lo:hi] = y + offs[:, k : k + 1]
    carry_ref[...] = offs[:, _K : _K + 1]


def kernel(x):
    m, n = x.shape
    grid = (n // _BLK,)

    r = np.arange(_SUB)
    tri = (r[:, None] <= r[None, :]).astype(np.float32)
    b = (np.arange(_BLK)[:, None] // _SUB < r[None, :]).astype(np.float32)
    tri = jnp.asarray(tri, dtype=jnp.bfloat16)
    b = jnp.asarray(b, dtype=jnp.bfloat16)

    return pl.pallas_call(
        _cumsum_body,
        grid=grid,
        in_specs=[
            pl.BlockSpec((m, _BLK), lambda i: (0, i)),
            pl.BlockSpec((_SUB, _SUB), lambda i: (0, 0)),
            pl.BlockSpec((_BLK, _SUB), lambda i: (0, 0)),
        ],
        out_specs=pl.BlockSpec((m, _BLK), lambda i: (0, i)),
        out_shape=jax.ShapeDtypeStruct((m, n), x.dtype),
        scratch_shapes=[pltpu.VMEM((m, 1), jnp.float32)],
    )(x, tri, b)


# speedup vs baseline: 10.1313x; 1.2351x over previous
import jax
import jax.numpy as jnp
import numpy as np
from jax import lax
from jax.experimental import pallas as pl
from jax.experimental.pallas import tpu as pltpu

_BLK = 4096
_SUB = 128
_K = _BLK // _SUB


def _cumsum_body(x_ref, tri_ref, b_ref, o_ref, carry_ref):
    i = pl.program_id(0)

    @pl.when(i == 0)
    def _init():
        carry_ref[...] = jnp.zeros_like(carry_ref)

    xb = x_ref[...].astype(jnp.bfloat16)
    pex = lax.dot(xb, b_ref[...], preferred_element_type=jnp.float32)
    offs = pex + carry_ref[...]
    for k in range(_K):
        lo, hi = k * _SUB, (k + 1) * _SUB
        y = lax.dot(
            xb[:, lo:hi], tri_ref[...], preferred_element_type=jnp.float32
        )
        o_ref[:, lo:hi] = y + offs[:, k : k + 1]
    carry_ref[...] = offs[:, _K : _K + 1]


def kernel(x):
    m, n = x.shape
    grid = (n // _BLK,)

    r = np.arange(_SUB)
    tri = (r[:, None] <= r[None, :]).astype(np.float32)
    b = (np.arange(_BLK)[:, None] // _SUB < r[None, :]).astype(np.float32)
    tri = jnp.asarray(tri, dtype=jnp.bfloat16)
    b = jnp.asarray(b, dtype=jnp.bfloat16)

    return pl.pallas_call(
        _cumsum_body,
        grid=grid,
        in_specs=[
            pl.BlockSpec((m, _BLK), lambda i: (0, i)),
            pl.BlockSpec((_SUB, _SUB), lambda i: (0, 0)),
            pl.BlockSpec((_BLK, _SUB), lambda i: (0, 0)),
        ],
        out_specs=pl.BlockSpec((m, _BLK), lambda i: (0, i)),
        out_shape=jax.ShapeDtypeStruct((m, n), x.dtype),
        scratch_shapes=[pltpu.VMEM((m, 1), jnp.float32)],
    )(x, tri, b)


# speedup vs baseline: 11.2190x; 1.1074x over previous
import jax
import jax.numpy as jnp
import numpy as np
from jax import lax
from jax.experimental import pallas as pl
from jax.experimental.pallas import tpu as pltpu

_BLK = 8192
_SUB = 128
_K = _BLK // _SUB


def _cumsum_body(x_ref, tri_ref, b_ref, o_ref, carry_ref):
    i = pl.program_id(0)

    @pl.when(i == 0)
    def _init():
        carry_ref[...] = jnp.zeros_like(carry_ref)

    xb = x_ref[...].astype(jnp.bfloat16)
    pex = lax.dot(xb, b_ref[...], preferred_element_type=jnp.float32)
    offs = pex + carry_ref[...]
    for k in range(_K):
        lo, hi = k * _SUB, (k + 1) * _SUB
        y = lax.dot(
            xb[:, lo:hi], tri_ref[...], preferred_element_type=jnp.float32
        )
        o_ref[:, lo:hi] = y + offs[:, k : k + 1]
    carry_ref[...] = offs[:, _K : _K + 1]


def kernel(x):
    m, n = x.shape
    grid = (n // _BLK,)

    r = np.arange(_SUB)
    tri = (r[:, None] <= r[None, :]).astype(np.float32)
    b = (np.arange(_BLK)[:, None] // _SUB < r[None, :]).astype(np.float32)
    tri = jnp.asarray(tri, dtype=jnp.bfloat16)
    b = jnp.asarray(b, dtype=jnp.bfloat16)

    return pl.pallas_call(
        _cumsum_body,
        grid=grid,
        in_specs=[
            pl.BlockSpec((m, _BLK), lambda i: (0, i)),
            pl.BlockSpec((_SUB, _SUB), lambda i: (0, 0)),
            pl.BlockSpec((_BLK, _SUB), lambda i: (0, 0)),
        ],
        out_specs=pl.BlockSpec((m, _BLK), lambda i: (0, i)),
        out_shape=jax.ShapeDtypeStruct((m, n), x.dtype),
        scratch_shapes=[pltpu.VMEM((m, 1), jnp.float32)],
    )(x, tri, b)
